# baseline (speedup 1.0000x reference)
import jax
import jax.numpy as jnp
from jax import lax
from jax.experimental import pallas as pl


def _moe_block(x_ref, Wf_ref, bf_ref, Wg_ref, bg_ref, out_ref):
    xb = x_ref[...]
    T, D = xb.shape
    E = Wg_ref.shape[0]
    logits = lax.dot_general(
        xb, Wg_ref[...], (((1,), (1,)), ((), ())),
        preferred_element_type=jnp.float32) + bg_ref[...][None, :]
    m = logits.max(axis=1, keepdims=True)
    ex = jnp.exp(logits - m)
    probs = ex / ex.sum(axis=1, keepdims=True)

    col = lax.broadcasted_iota(jnp.int32, probs.shape, 1)
    i1 = jnp.argmax(probs, axis=1)[:, None]
    v1 = jnp.max(probs, axis=1, keepdims=True)
    masked = jnp.where(col == i1, -jnp.inf, probs)
    i2 = jnp.argmax(masked, axis=1)[:, None]
    v2 = jnp.max(masked, axis=1, keepdims=True)
    gate = jnp.where(col == i1, v1, jnp.where(col == i2, v2, 0.0))

    h = lax.dot_general(
        xb, Wf_ref[...], (((1,), (1,)), ((), ())),
        preferred_element_type=jnp.float32) + bf_ref[...]
    h = jnp.maximum(h, 0.0).astype(jnp.bfloat16)
    hr = h.reshape(T, E, D)
    out_ref[...] = (gate[:, :, None] * hr.astype(jnp.float32)).sum(axis=1)


@jax.jit
def kernel(x, W, b, Wg, bg):
    N, D = x.shape
    E = W.shape[0]
    Wf = W.reshape(E * D, D)
    bf = b.reshape(1, E * D)
    T = 512
    grid = (N // T,)
    return pl.pallas_call(
        _moe_block,
        grid=grid,
        in_specs=[
            pl.BlockSpec((T, D), lambda i: (i, 0)),
            pl.BlockSpec((E * D, D), lambda i: (0, 0)),
            pl.BlockSpec((1, E * D), lambda i: (0, 0)),
            pl.BlockSpec((E, D), lambda i: (0, 0)),
            pl.BlockSpec((E,), lambda i: (0,)),
        ],
        out_specs=pl.BlockSpec((T, D), lambda i: (i, 0)),
        out_shape=jax.ShapeDtypeStruct((N, D), x.dtype),
    )(x, Wf, bf, Wg, bg)


# speedup vs baseline: 1.5469x; 1.5469x over previous
import jax
import jax.numpy as jnp
from jax import lax
from jax.experimental import pallas as pl


def _moe_block(x_ref, Wf_ref, bf_ref, Wg_ref, bg_ref, out_ref):
    xb = x_ref[...]
    T, D = xb.shape
    E = Wg_ref.shape[0]
    logits = lax.dot_general(
        xb, Wg_ref[...], (((1,), (1,)), ((), ())),
        preferred_element_type=jnp.float32) + bg_ref[...][None, :]
    m = logits.max(axis=1, keepdims=True)
    ex = jnp.exp(logits - m)
    probs = ex / ex.sum(axis=1, keepdims=True)

    col = lax.broadcasted_iota(jnp.int32, probs.shape, 1)
    i1 = jnp.argmax(probs, axis=1)[:, None]
    v1 = jnp.max(probs, axis=1, keepdims=True)
    masked = jnp.where(col == i1, -jnp.inf, probs)
    i2 = jnp.argmax(masked, axis=1)[:, None]
    v2 = jnp.max(masked, axis=1, keepdims=True)
    gate = jnp.where(col == i1, v1, jnp.where(col == i2, v2, 0.0))

    h = lax.dot_general(
        xb, Wf_ref[...], (((1,), (1,)), ((), ())),
        preferred_element_type=jnp.float32) + bf_ref[...]
    h = jnp.maximum(h, 0.0).astype(jnp.bfloat16)
    acc = jnp.zeros((T, D), jnp.float32)
    for e in range(E):
        acc = acc + gate[:, e][:, None] * h[:, e * D:(e + 1) * D].astype(
            jnp.float32)
    out_ref[...] = acc


@jax.jit
def kernel(x, W, b, Wg, bg):
    N, D = x.shape
    E = W.shape[0]
    Wf = W.reshape(E * D, D)
    bf = b.reshape(1, E * D)
    T = 512
    grid = (N // T,)
    return pl.pallas_call(
        _moe_block,
        grid=grid,
        in_specs=[
            pl.BlockSpec((T, D), lambda i: (i, 0)),
            pl.BlockSpec((E * D, D), lambda i: (0, 0)),
            pl.BlockSpec((1, E * D), lambda i: (0, 0)),
            pl.BlockSpec((E, D), lambda i: (0, 0)),
            pl.BlockSpec((E,), lambda i: (0,)),
        ],
        out_specs=pl.BlockSpec((T, D), lambda i: (i, 0)),
        out_shape=jax.ShapeDtypeStruct((N, D), x.dtype),
    )(x, Wf, bf, Wg, bg)


# speedup vs baseline: 1.7601x; 1.1378x over previous
import functools

import jax
import jax.numpy as jnp
from jax.experimental import pallas as pl


def _moe_block(x_ref, W_ref, b_ref, Wg_ref, bg_ref, out_ref):
    xb = x_ref[...]
    logits = jax.lax.dot_general(
        xb, Wg_ref[...], (((1,), (1,)), ((), ())),
        preferred_element_type=jnp.float32) + bg_ref[...][None, :]
    m = logits.max(axis=1, keepdims=True)
    ex = jnp.exp(logits - m)
    probs = ex / ex.sum(axis=1, keepdims=True)

    E = probs.shape[1]
    col = jax.lax.broadcasted_iota(jnp.int32, probs.shape, 1)
    i1 = jnp.argmax(probs, axis=1)[:, None]
    v1 = jnp.max(probs, axis=1, keepdims=True)
    masked = jnp.where(col == i1, -jnp.inf, probs)
    i2 = jnp.argmax(masked, axis=1)[:, None]
    v2 = jnp.max(masked, axis=1, keepdims=True)
    gate = jnp.where(col == i1, v1, jnp.where(col == i2, v2, 0.0))

    xb16 = xb.astype(jnp.bfloat16)
    acc = jnp.zeros_like(xb)
    for e in range(E):
        h = jax.lax.dot_general(
            xb16, W_ref[e].astype(jnp.bfloat16), (((1,), (1,)), ((), ())),
            preferred_element_type=jnp.float32) + b_ref[e][None, :]
        acc = acc + gate[:, e][:, None] * jnp.maximum(h, 0.0)
    out_ref[...] = acc


@jax.jit
def kernel(x, W, b, Wg, bg):
    N, D = x.shape
    E = W.shape[0]
    T = 512
    grid = (N // T,)
    return pl.pallas_call(
        _moe_block,
        grid=grid,
        in_specs=[
            pl.BlockSpec((T, D), lambda i: (i, 0)),
            pl.BlockSpec((E, D, D), lambda i: (0, 0, 0)),
            pl.BlockSpec((E, D), lambda i: (0, 0)),
            pl.BlockSpec((E, D), lambda i: (0, 0)),
            pl.BlockSpec((E,), lambda i: (0,)),
        ],
        out_specs=pl.BlockSpec((T, D), lambda i: (i, 0)),
        out_shape=jax.ShapeDtypeStruct((N, D), x.dtype),
    )(x, W, b, Wg, bg)
